# baseline (speedup 1.0000x reference)
import functools

import jax
import jax.numpy as jnp
from jax import lax
from jax.experimental import pallas as pl
from jax.experimental.pallas import tpu as pltpu
from jax.experimental.pallas import tpu_sc as plsc

N_NODES = 10000
E_EDGES = 320000
D = 128
G = 128

NC = 2
NS = 16
NW = NC * NS
N_PAD = 10240
RPT_ACC = N_PAD // NS
CHUNK = 128
EROWS = E_EDGES // CHUNK
BIG = 24
MAXR = 80

_MESH = plsc.VectorSubcoreMesh(core_axis_name="c", subcore_axis_name="s")


def _fill_1d(ref, n, val):
  def body(i, carry):
    ref[pl.ds(i * 16, 16)] = jnp.full((16,), val, jnp.float32)
    return carry
  lax.fori_loop(0, n // 16, body, 0)


def _zero_2d(ref, rows):
  def body(i, carry):
    for k in range(D // 16):
      ref[i, pl.ds(k * 16, 16)] = jnp.zeros((16,), jnp.float32)
    return carry
  lax.fori_loop(0, rows, body, 0)


def _edge_assign(wid):
  rbase = 8 * (wid * 9 + jnp.minimum(wid, BIG))
  ntot = jnp.where(wid < BIG, 80, jnp.where(wid == NW - 1, 76, 72))
  return rbase, ntot


def _preload_rows(eid_hbm, rbase, wid, buf, sem):
  pltpu.async_copy(eid_hbm.at[pl.ds(rbase, 72)], buf.at[pl.ds(0, 72)], sem)

  @pl.when(wid < BIG)
  def _more8():
    pltpu.async_copy(eid_hbm.at[pl.ds(rbase + 72, 8)],
                     buf.at[pl.ds(72, 8)], sem)

  @pl.when(wid == NW - 1)
  def _more4():
    pltpu.async_copy(eid_hbm.at[pl.ds(rbase + 72, 4)],
                     buf.at[pl.ds(72, 4)], sem)


def _wait_rows(eid_hbm, rbase, wid, buf, sem):
  pltpu.make_async_copy(eid_hbm.at[pl.ds(rbase, 72)],
                        buf.at[pl.ds(0, 72)], sem).wait()

  @pl.when(wid < BIG)
  def _more8():
    pltpu.make_async_copy(eid_hbm.at[pl.ds(rbase + 72, 8)],
                          buf.at[pl.ds(72, 8)], sem).wait()

  @pl.when(wid == NW - 1)
  def _more4():
    pltpu.make_async_copy(eid_hbm.at[pl.ds(rbase + 72, 4)],
                          buf.at[pl.ds(72, 4)], sem).wait()


@functools.partial(
    pl.kernel,
    mesh=_MESH,
    out_type=jax.ShapeDtypeStruct((NC * N_PAD,), jnp.float32),
    scratch_types=[
        pltpu.VMEM((MAXR, CHUNK), jnp.int32),
        pltpu.VMEM((CHUNK,), jnp.float32),
        pltpu.VMEM((RPT_ACC,), jnp.float32),
        pltpu.VMEM_SHARED((N_PAD,), jnp.float32),
        pltpu.SemaphoreType.DMA,
        pltpu.SemaphoreType.DMA,
    ],
)
def _deg_kernel(eid_hbm, out_hbm, didx, ones_v, buf_v, acc_sh, sem_i, sem_s):
  c = lax.axis_index("c")
  s = lax.axis_index("s")
  wid = c * NS + s
  rbase, ntot = _edge_assign(wid)
  _preload_rows(eid_hbm, rbase, wid, didx, sem_i)
  _fill_1d(ones_v, CHUNK, 1.0)
  _fill_1d(buf_v, RPT_ACC, 0.0)
  pltpu.sync_copy(buf_v, acc_sh.at[pl.ds(s * RPT_ACC, RPT_ACC)])
  _wait_rows(eid_hbm, rbase, wid, didx, sem_i)
  plsc.subcore_barrier()

  def fire(j, carry):
    pltpu.async_copy(ones_v, acc_sh.at[didx.at[j]], sem_s, add=True)
    return carry
  lax.fori_loop(0, ntot, fire, 0)

  def drain(j, carry):
    pltpu.make_async_copy(ones_v, acc_sh.at[didx.at[0]], sem_s).wait()
    return carry
  lax.fori_loop(0, ntot, drain, 0)

  plsc.subcore_barrier()
  pltpu.sync_copy(acc_sh.at[pl.ds(s * RPT_ACC, RPT_ACC)], buf_v)
  pltpu.sync_copy(buf_v,
                  out_hbm.at[pl.ds(c * N_PAD + s * RPT_ACC, RPT_ACC)])


PH_A = 56


@functools.partial(
    pl.kernel,
    mesh=_MESH,
    out_type=jax.ShapeDtypeStruct((NC, N_PAD, D), jnp.float32),
    scratch_types=[
        pltpu.VMEM((PH_A * CHUNK,), jnp.int32),
        pltpu.VMEM((PH_A, CHUNK), jnp.int32),
        pltpu.VMEM((CHUNK, D), jnp.float32),
        pltpu.VMEM((CHUNK, D), jnp.float32),
        pltpu.VMEM_SHARED((N_PAD, D), jnp.float32),
        pltpu.SemaphoreType.DMA,
        pltpu.SemaphoreType.DMA,
        pltpu.SemaphoreType.DMA,
        pltpu.SemaphoreType.DMA,
    ],
)
def _msg_kernel(ei_hbm, eid_hbm, hp_hbm, out_hbm, sidx, didx, rows0, rows1,
                acc_sh, sem_i, sem_g0, sem_g1, sem_d):
  c = lax.axis_index("c")
  s = lax.axis_index("s")
  wid = c * NS + s
  rbase, ntot = _edge_assign(wid)
  rows = (rows0, rows1)
  sems = (sem_g0, sem_g1)

  def gather(j, b):
    pltpu.async_copy(hp_hbm.at[sidx.at[pl.ds(j * CHUNK, CHUNK)]], rows[b],
                     sems[b])

  def pipe(n, prefired):
    if not prefired:
      gather(0, 0)
      gather(1, 1)

    def step(j, b, fire):
      pltpu.make_async_copy(hp_hbm.at[sidx.at[pl.ds(0, CHUNK)]], rows[b],
                            sems[b]).wait()
      pltpu.sync_copy(rows[b], acc_sh.at[didx.at[j]], add=True)
      if fire:
        gather(j + 2, b)

    def outer(i, carry):
      step(2 * i, 0, True)
      step(2 * i + 1, 1, True)
      return carry
    lax.fori_loop(0, (n - 2) // 2, outer, 0)
    for b in range(2):
      step(n - 2 + b, b, False)

  pltpu.async_copy(ei_hbm.at[0, pl.ds(rbase * CHUNK, PH_A * CHUNK)], sidx,
                   sem_i)
  pltpu.async_copy(eid_hbm.at[pl.ds(rbase, PH_A)], didx, sem_i)
  _zero_2d(rows0, CHUNK)
  r0 = s * RPT_ACC
  for k in range(RPT_ACC // CHUNK):
    pltpu.async_copy(rows0, acc_sh.at[pl.ds(r0 + k * CHUNK, CHUNK)], sem_d)
  for k in range(RPT_ACC // CHUNK):
    pltpu.make_async_copy(rows0, acc_sh.at[pl.ds(r0, CHUNK)], sem_d).wait()
  pltpu.make_async_copy(ei_hbm.at[0, pl.ds(rbase * CHUNK, PH_A * CHUNK)],
                        sidx, sem_i).wait()
  pltpu.make_async_copy(eid_hbm.at[pl.ds(rbase, PH_A)], didx, sem_i).wait()
  gather(0, 0)
  gather(1, 1)
  plsc.subcore_barrier()

  pipe(PH_A, prefired=True)

  rb2 = rbase + PH_A
  pltpu.async_copy(ei_hbm.at[0, pl.ds(rb2 * CHUNK, 16 * CHUNK)],
                   sidx.at[pl.ds(0, 16 * CHUNK)], sem_i)
  pltpu.async_copy(eid_hbm.at[pl.ds(rb2, 16)], didx.at[pl.ds(0, 16)], sem_i)

  @pl.when(wid < BIG)
  def _more8():
    pltpu.sync_copy(ei_hbm.at[0, pl.ds((rb2 + 16) * CHUNK, 8 * CHUNK)],
                    sidx.at[pl.ds(16 * CHUNK, 8 * CHUNK)])
    pltpu.sync_copy(eid_hbm.at[pl.ds(rb2 + 16, 8)], didx.at[pl.ds(16, 8)])

  @pl.when(wid == NW - 1)
  def _more4():
    pltpu.sync_copy(ei_hbm.at[0, pl.ds((rb2 + 16) * CHUNK, 4 * CHUNK)],
                    sidx.at[pl.ds(16 * CHUNK, 4 * CHUNK)])
    pltpu.sync_copy(eid_hbm.at[pl.ds(rb2 + 16, 4)], didx.at[pl.ds(16, 4)])

  pltpu.make_async_copy(ei_hbm.at[0, pl.ds(rb2 * CHUNK, 16 * CHUNK)],
                        sidx.at[pl.ds(0, 16 * CHUNK)], sem_i).wait()
  pltpu.make_async_copy(eid_hbm.at[pl.ds(rb2, 16)], didx.at[pl.ds(0, 16)],
                        sem_i).wait()
  pipe(ntot - PH_A, prefired=False)

  plsc.subcore_barrier()

  for k in range(RPT_ACC // CHUNK):
    b = k % 2
    if k >= 2:
      pltpu.make_async_copy(rows[b], out_hbm.at[c, pl.ds(0, CHUNK)],
                            sem_d).wait()
    pltpu.sync_copy(acc_sh.at[pl.ds(r0 + k * CHUNK, CHUNK)], rows[b])
    pltpu.async_copy(rows[b], out_hbm.at[c, pl.ds(r0 + k * CHUNK, CHUNK)],
                     sem_d)
  for b in range(2):
    pltpu.make_async_copy(rows[b], out_hbm.at[c, pl.ds(0, CHUNK)],
                          sem_d).wait()


def _k2_body(x_ref, w_ref, d0_ref, d1_ref, hp_ref, dinv_ref):
  h = jnp.dot(x_ref[...], w_ref[...],
              preferred_element_type=jnp.float32,
              precision=lax.Precision.HIGHEST)
  deg = d0_ref[...] + d1_ref[...] + 1.0
  dinv = lax.rsqrt(deg)
  hp_ref[...] = h * dinv.reshape(N_NODES, 1)
  dinv_ref[...] = dinv


def _k2_call(x, W1, d0, d1):
  return pl.pallas_call(
      _k2_body,
      out_shape=[
          jax.ShapeDtypeStruct((N_NODES, D), jnp.float32),
          jax.ShapeDtypeStruct((N_NODES,), jnp.float32),
      ],
  )(x, W1, d0, d1)


def _k4_body(acc_ref, hp_ref, dinv_ref, batch_ref, b1_ref, wl_ref, bl_ref,
             out_ref):
  acc = (acc_ref[0, pl.ds(0, N_NODES), :] +
         acc_ref[1, pl.ds(0, N_NODES), :])
  node = dinv_ref[...].reshape(N_NODES, 1) * (acc + hp_ref[...])
  node = jnp.maximum(node + b1_ref[...], 0.0)
  onehot_t = (batch_ref[...].reshape(1, N_NODES) ==
              lax.broadcasted_iota(jnp.int32, (G, 1), 0)).astype(jnp.float32)
  sums = lax.dot_general(
      onehot_t, node, (((1,), (0,)), ((), ())),
      preferred_element_type=jnp.float32, precision=lax.Precision.HIGHEST)
  cnt = lax.dot_general(
      onehot_t, jnp.ones((N_NODES, 1), jnp.float32), (((1,), (0,)), ((), ())),
      preferred_element_type=jnp.float32, precision=lax.Precision.HIGHEST)
  pooled = sums / jnp.maximum(cnt, 1.0)
  out_ref[...] = jnp.dot(pooled, wl_ref[...],
                         preferred_element_type=jnp.float32,
                         precision=lax.Precision.HIGHEST) + bl_ref[...]


def _k4_call(acc, hp, dinv, batch1d, b1, Wl, bl):
  return pl.pallas_call(
      _k4_body,
      out_shape=jax.ShapeDtypeStruct((G, D), jnp.float32),
  )(acc, hp, dinv, batch1d, b1, Wl, bl)


@jax.jit
def kernel(x, edge_index, batch, W1, b1, Wl, bl):
  eid3 = edge_index[1].reshape(EROWS, CHUNK)
  deg_flat = _deg_kernel(eid3)
  d0 = deg_flat[:N_NODES]
  d1 = deg_flat[N_PAD:N_PAD + N_NODES]
  hp, dinv = _k2_call(x, W1, d0, d1)
  acc = _msg_kernel(edge_index, eid3, hp)
  out = _k4_call(acc, hp, dinv, batch,
                 b1.reshape(1, D), Wl, bl.reshape(1, D))
  return out


# speedup vs baseline: 1.0685x; 1.0685x over previous
import functools

import jax
import jax.numpy as jnp
from jax import lax
from jax.experimental import pallas as pl
from jax.experimental.pallas import tpu as pltpu
from jax.experimental.pallas import tpu_sc as plsc

N_NODES = 10000
E_EDGES = 320000
D = 128
G = 128

NC = 2
NS = 16
NW = NC * NS
N_PAD = 10240
RPT_ACC = N_PAD // NS
CHUNK = 128
EROWS = E_EDGES // CHUNK
BIG = 24
MAXR = 80

_MESH = plsc.VectorSubcoreMesh(core_axis_name="c", subcore_axis_name="s")


def _fill_1d(ref, n, val):
  def body(i, carry):
    ref[pl.ds(i * 16, 16)] = jnp.full((16,), val, jnp.float32)
    return carry
  lax.fori_loop(0, n // 16, body, 0)


def _zero_2d(ref, rows):
  def body(i, carry):
    for k in range(D // 16):
      ref[i, pl.ds(k * 16, 16)] = jnp.zeros((16,), jnp.float32)
    return carry
  lax.fori_loop(0, rows, body, 0)


def _edge_assign(wid):
  rbase = 8 * (wid * 9 + jnp.minimum(wid, BIG))
  ntot = jnp.where(wid < BIG, 80, jnp.where(wid == NW - 1, 76, 72))
  return rbase, ntot


def _preload_rows(ei3_hbm, which, rbase, wid, buf, sem):
  pltpu.async_copy(ei3_hbm.at[which, pl.ds(rbase, 72)],
                   buf.at[pl.ds(0, 72)], sem)

  @pl.when(wid < BIG)
  def _more8():
    pltpu.async_copy(ei3_hbm.at[which, pl.ds(rbase + 72, 8)],
                     buf.at[pl.ds(72, 8)], sem)

  @pl.when(wid == NW - 1)
  def _more4():
    pltpu.async_copy(ei3_hbm.at[which, pl.ds(rbase + 72, 4)],
                     buf.at[pl.ds(72, 4)], sem)


def _wait_rows(ei3_hbm, which, rbase, wid, buf, sem):
  pltpu.make_async_copy(ei3_hbm.at[which, pl.ds(rbase, 72)],
                        buf.at[pl.ds(0, 72)], sem).wait()

  @pl.when(wid < BIG)
  def _more8():
    pltpu.make_async_copy(ei3_hbm.at[which, pl.ds(rbase + 72, 8)],
                          buf.at[pl.ds(72, 8)], sem).wait()

  @pl.when(wid == NW - 1)
  def _more4():
    pltpu.make_async_copy(ei3_hbm.at[which, pl.ds(rbase + 72, 4)],
                          buf.at[pl.ds(72, 4)], sem).wait()


@functools.partial(
    pl.kernel,
    mesh=_MESH,
    out_type=jax.ShapeDtypeStruct((NC * N_PAD,), jnp.float32),
    scratch_types=[
        pltpu.VMEM((MAXR, CHUNK), jnp.int32),
        pltpu.VMEM((CHUNK,), jnp.float32),
        pltpu.VMEM((RPT_ACC,), jnp.float32),
        pltpu.VMEM_SHARED((N_PAD,), jnp.float32),
        pltpu.SemaphoreType.DMA,
        pltpu.SemaphoreType.DMA,
    ],
)
def _deg_kernel(ei3_hbm, out_hbm, didx, ones_v, buf_v, acc_sh, sem_i, sem_s):
  c = lax.axis_index("c")
  s = lax.axis_index("s")
  wid = c * NS + s
  rbase, ntot = _edge_assign(wid)
  _preload_rows(ei3_hbm, 1, rbase, wid, didx, sem_i)
  _fill_1d(ones_v, CHUNK, 1.0)
  _fill_1d(buf_v, RPT_ACC, 0.0)
  pltpu.sync_copy(buf_v, acc_sh.at[pl.ds(s * RPT_ACC, RPT_ACC)])
  _wait_rows(ei3_hbm, 1, rbase, wid, didx, sem_i)
  plsc.subcore_barrier()

  def fire(j, carry):
    pltpu.async_copy(ones_v, acc_sh.at[didx.at[j]], sem_s, add=True)
    return carry
  lax.fori_loop(0, ntot, fire, 0)

  def drain(j, carry):
    pltpu.make_async_copy(ones_v, acc_sh.at[didx.at[0]], sem_s).wait()
    return carry
  lax.fori_loop(0, ntot, drain, 0)

  plsc.subcore_barrier()
  pltpu.sync_copy(acc_sh.at[pl.ds(s * RPT_ACC, RPT_ACC)], buf_v)
  pltpu.sync_copy(buf_v,
                  out_hbm.at[pl.ds(c * N_PAD + s * RPT_ACC, RPT_ACC)])


PH_A = 56


@functools.partial(
    pl.kernel,
    mesh=_MESH,
    out_type=jax.ShapeDtypeStruct((NC, N_PAD, D), jnp.float32),
    scratch_types=[
        pltpu.VMEM((PH_A, CHUNK), jnp.int32),
        pltpu.VMEM((PH_A, CHUNK), jnp.int32),
        pltpu.VMEM((CHUNK, D), jnp.float32),
        pltpu.VMEM((CHUNK, D), jnp.float32),
        pltpu.VMEM_SHARED((N_PAD, D), jnp.float32),
        pltpu.SemaphoreType.DMA,
        pltpu.SemaphoreType.DMA,
        pltpu.SemaphoreType.DMA,
        pltpu.SemaphoreType.DMA,
    ],
)
def _msg_kernel(ei3_hbm, hp_hbm, out_hbm, sidx, didx, rows0, rows1,
                acc_sh, sem_i, sem_g0, sem_g1, sem_d):
  c = lax.axis_index("c")
  s = lax.axis_index("s")
  wid = c * NS + s
  rbase, ntot = _edge_assign(wid)
  rows = (rows0, rows1)
  sems = (sem_g0, sem_g1)

  def gather(j, b):
    pltpu.async_copy(hp_hbm.at[sidx.at[j]], rows[b], sems[b])

  def pipe(n, prefired):
    if not prefired:
      gather(0, 0)
      gather(1, 1)

    def step(j, b, fire):
      pltpu.make_async_copy(hp_hbm.at[sidx.at[j]], rows[b], sems[b]).wait()
      pltpu.sync_copy(rows[b], acc_sh.at[didx.at[j]], add=True)
      if fire:
        gather(j + 2, b)

    def outer(i, carry):
      step(2 * i, 0, True)
      step(2 * i + 1, 1, True)
      return carry
    lax.fori_loop(0, (n - 2) // 2, outer, 0)
    for b in range(2):
      step(n - 2 + b, b, False)

  pltpu.async_copy(ei3_hbm.at[0, pl.ds(rbase, PH_A)], sidx, sem_i)
  pltpu.async_copy(ei3_hbm.at[1, pl.ds(rbase, PH_A)], didx, sem_i)
  _zero_2d(rows0, CHUNK)
  r0 = s * RPT_ACC
  for k in range(RPT_ACC // CHUNK):
    pltpu.async_copy(rows0, acc_sh.at[pl.ds(r0 + k * CHUNK, CHUNK)], sem_d)
  for k in range(RPT_ACC // CHUNK):
    pltpu.make_async_copy(rows0, acc_sh.at[pl.ds(r0, CHUNK)], sem_d).wait()
  pltpu.make_async_copy(ei3_hbm.at[0, pl.ds(rbase, PH_A)], sidx, sem_i).wait()
  pltpu.make_async_copy(ei3_hbm.at[1, pl.ds(rbase, PH_A)], didx, sem_i).wait()
  gather(0, 0)
  gather(1, 1)
  plsc.subcore_barrier()

  pipe(PH_A, prefired=True)

  rb2 = rbase + PH_A
  pltpu.async_copy(ei3_hbm.at[0, pl.ds(rb2, 16)], sidx.at[pl.ds(0, 16)],
                   sem_i)
  pltpu.async_copy(ei3_hbm.at[1, pl.ds(rb2, 16)], didx.at[pl.ds(0, 16)],
                   sem_i)

  @pl.when(wid < BIG)
  def _more8():
    pltpu.sync_copy(ei3_hbm.at[0, pl.ds(rb2 + 16, 8)], sidx.at[pl.ds(16, 8)])
    pltpu.sync_copy(ei3_hbm.at[1, pl.ds(rb2 + 16, 8)], didx.at[pl.ds(16, 8)])

  @pl.when(wid == NW - 1)
  def _more4():
    pltpu.sync_copy(ei3_hbm.at[0, pl.ds(rb2 + 16, 4)], sidx.at[pl.ds(16, 4)])
    pltpu.sync_copy(ei3_hbm.at[1, pl.ds(rb2 + 16, 4)], didx.at[pl.ds(16, 4)])

  pltpu.make_async_copy(ei3_hbm.at[0, pl.ds(rb2, 16)], sidx.at[pl.ds(0, 16)],
                        sem_i).wait()
  pltpu.make_async_copy(ei3_hbm.at[1, pl.ds(rb2, 16)], didx.at[pl.ds(0, 16)],
                        sem_i).wait()
  pipe(ntot - PH_A, prefired=False)

  plsc.subcore_barrier()

  for k in range(RPT_ACC // CHUNK):
    b = k % 2
    if k >= 2:
      pltpu.make_async_copy(rows[b], out_hbm.at[c, pl.ds(0, CHUNK)],
                            sem_d).wait()
    pltpu.sync_copy(acc_sh.at[pl.ds(r0 + k * CHUNK, CHUNK)], rows[b])
    pltpu.async_copy(rows[b], out_hbm.at[c, pl.ds(r0 + k * CHUNK, CHUNK)],
                     sem_d)
  for b in range(2):
    pltpu.make_async_copy(rows[b], out_hbm.at[c, pl.ds(0, CHUNK)],
                          sem_d).wait()


def _k2_body(x_ref, w_ref, d0_ref, d1_ref, hp_ref, dinv_ref):
  h = jnp.dot(x_ref[...], w_ref[...],
              preferred_element_type=jnp.float32,
              precision=lax.Precision.HIGHEST)
  deg = d0_ref[...] + d1_ref[...] + 1.0
  dinv = lax.rsqrt(deg)
  hp_ref[...] = h * dinv.reshape(N_NODES, 1)
  dinv_ref[...] = dinv


def _k2_call(x, W1, d0, d1):
  return pl.pallas_call(
      _k2_body,
      out_shape=[
          jax.ShapeDtypeStruct((N_NODES, D), jnp.float32),
          jax.ShapeDtypeStruct((N_NODES,), jnp.float32),
      ],
  )(x, W1, d0, d1)


def _k4_body(acc_ref, hp_ref, dinv_ref, batch_ref, b1_ref, wl_ref, bl_ref,
             out_ref):
  acc = (acc_ref[0, pl.ds(0, N_NODES), :] +
         acc_ref[1, pl.ds(0, N_NODES), :])
  node = dinv_ref[...].reshape(N_NODES, 1) * (acc + hp_ref[...])
  node = jnp.maximum(node + b1_ref[...], 0.0)
  onehot_t = (batch_ref[...].reshape(1, N_NODES) ==
              lax.broadcasted_iota(jnp.int32, (G, 1), 0)).astype(jnp.float32)
  sums = lax.dot_general(
      onehot_t, node, (((1,), (0,)), ((), ())),
      preferred_element_type=jnp.float32, precision=lax.Precision.HIGHEST)
  cnt = lax.dot_general(
      onehot_t, jnp.ones((N_NODES, 1), jnp.float32), (((1,), (0,)), ((), ())),
      preferred_element_type=jnp.float32, precision=lax.Precision.HIGHEST)
  pooled = sums / jnp.maximum(cnt, 1.0)
  out_ref[...] = jnp.dot(pooled, wl_ref[...],
                         preferred_element_type=jnp.float32,
                         precision=lax.Precision.HIGHEST) + bl_ref[...]


def _k4_call(acc, hp, dinv, batch1d, b1, Wl, bl):
  return pl.pallas_call(
      _k4_body,
      out_shape=jax.ShapeDtypeStruct((G, D), jnp.float32),
  )(acc, hp, dinv, batch1d, b1, Wl, bl)


@jax.jit
def kernel(x, edge_index, batch, W1, b1, Wl, bl):
  ei3 = edge_index.reshape(2, EROWS, CHUNK)
  deg_flat = _deg_kernel(ei3)
  d0 = deg_flat[:N_NODES]
  d1 = deg_flat[N_PAD:N_PAD + N_NODES]
  hp, dinv = _k2_call(x, W1, d0, d1)
  acc = _msg_kernel(ei3, hp)
  out = _k4_call(acc, hp, dinv, batch,
                 b1.reshape(1, D), Wl, bl.reshape(1, D))
  return out


# speedup vs baseline: 1.0796x; 1.0104x over previous
import functools

import jax
import jax.numpy as jnp
from jax import lax
from jax.experimental import pallas as pl
from jax.experimental.pallas import tpu as pltpu
from jax.experimental.pallas import tpu_sc as plsc

N_NODES = 10000
E_EDGES = 320000
D = 128
G = 128

NC = 2
NS = 16
NW = NC * NS
N_PAD = 10240
RPT_ACC = N_PAD // NS
CHUNK = 128
EROWS = E_EDGES // CHUNK
BIG = 24
MAXR = 80

_MESH = plsc.VectorSubcoreMesh(core_axis_name="c", subcore_axis_name="s")


def _fill_1d(ref, n, val):
  def body(i, carry):
    ref[pl.ds(i * 16, 16)] = jnp.full((16,), val, jnp.float32)
    return carry
  lax.fori_loop(0, n // 16, body, 0)


def _zero_2d(ref, rows):
  def body(i, carry):
    for k in range(D // 16):
      ref[i, pl.ds(k * 16, 16)] = jnp.zeros((16,), jnp.float32)
    return carry
  lax.fori_loop(0, rows, body, 0)


def _edge_assign(wid):
  rbase = 8 * (wid * 9 + jnp.minimum(wid, BIG))
  ntot = jnp.where(wid < BIG, 80, jnp.where(wid == NW - 1, 76, 72))
  return rbase, ntot


def _preload_rows(ei3_hbm, which, rbase, wid, buf, sem):
  pltpu.async_copy(ei3_hbm.at[which, pl.ds(rbase, 72)],
                   buf.at[pl.ds(0, 72)], sem)

  @pl.when(wid < BIG)
  def _more8():
    pltpu.async_copy(ei3_hbm.at[which, pl.ds(rbase + 72, 8)],
                     buf.at[pl.ds(72, 8)], sem)

  @pl.when(wid == NW - 1)
  def _more4():
    pltpu.async_copy(ei3_hbm.at[which, pl.ds(rbase + 72, 4)],
                     buf.at[pl.ds(72, 4)], sem)


def _wait_rows(ei3_hbm, which, rbase, wid, buf, sem):
  pltpu.make_async_copy(ei3_hbm.at[which, pl.ds(rbase, 72)],
                        buf.at[pl.ds(0, 72)], sem).wait()

  @pl.when(wid < BIG)
  def _more8():
    pltpu.make_async_copy(ei3_hbm.at[which, pl.ds(rbase + 72, 8)],
                          buf.at[pl.ds(72, 8)], sem).wait()

  @pl.when(wid == NW - 1)
  def _more4():
    pltpu.make_async_copy(ei3_hbm.at[which, pl.ds(rbase + 72, 4)],
                          buf.at[pl.ds(72, 4)], sem).wait()


@functools.partial(
    pl.kernel,
    mesh=_MESH,
    out_type=jax.ShapeDtypeStruct((NC * N_PAD,), jnp.float32),
    scratch_types=[
        pltpu.VMEM((MAXR, CHUNK), jnp.int32),
        pltpu.VMEM((CHUNK,), jnp.float32),
        pltpu.VMEM((RPT_ACC,), jnp.float32),
        pltpu.VMEM_SHARED((N_PAD,), jnp.float32),
        pltpu.SemaphoreType.DMA,
        pltpu.SemaphoreType.DMA,
    ],
)
def _deg_kernel(ei3_hbm, out_hbm, didx, ones_v, buf_v, acc_sh, sem_i, sem_s):
  c = lax.axis_index("c")
  s = lax.axis_index("s")
  wid = c * NS + s
  rbase, ntot = _edge_assign(wid)
  _preload_rows(ei3_hbm, 1, rbase, wid, didx, sem_i)
  _fill_1d(ones_v, CHUNK, 1.0)
  _fill_1d(buf_v, RPT_ACC, 0.0)
  pltpu.sync_copy(buf_v, acc_sh.at[pl.ds(s * RPT_ACC, RPT_ACC)])
  _wait_rows(ei3_hbm, 1, rbase, wid, didx, sem_i)
  plsc.subcore_barrier()

  def fire(j, carry):
    pltpu.async_copy(ones_v, acc_sh.at[didx.at[j]], sem_s, add=True)
    return carry
  lax.fori_loop(0, ntot, fire, 0)

  def drain(j, carry):
    pltpu.make_async_copy(ones_v, acc_sh.at[didx.at[0]], sem_s).wait()
    return carry
  lax.fori_loop(0, ntot, drain, 0)

  plsc.subcore_barrier()
  pltpu.sync_copy(acc_sh.at[pl.ds(s * RPT_ACC, RPT_ACC)], buf_v)
  pltpu.sync_copy(buf_v,
                  out_hbm.at[pl.ds(c * N_PAD + s * RPT_ACC, RPT_ACC)])


PH_A = 56


@functools.partial(
    pl.kernel,
    mesh=_MESH,
    out_type=jax.ShapeDtypeStruct((NC, N_PAD, D), jnp.float32),
    scratch_types=[
        pltpu.VMEM((PH_A, CHUNK), jnp.int32),
        pltpu.VMEM((PH_A, CHUNK), jnp.int32),
        pltpu.VMEM((CHUNK, D), jnp.float32),
        pltpu.VMEM((CHUNK, D), jnp.float32),
        pltpu.VMEM_SHARED((N_PAD, D), jnp.float32),
        pltpu.SemaphoreType.DMA,
        pltpu.SemaphoreType.DMA,
        pltpu.SemaphoreType.DMA,
        pltpu.SemaphoreType.DMA,
    ],
)
def _msg_kernel(ei3_hbm, hp_hbm, out_hbm, sidx, didx, rows0, rows1,
                acc_sh, sem_i, sem_g0, sem_g1, sem_d):
  c = lax.axis_index("c")
  s = lax.axis_index("s")
  wid = c * NS + s
  rbase, ntot = _edge_assign(wid)
  rows = (rows0, rows1)
  sems = (sem_g0, sem_g1)

  def gather(j, b):
    pltpu.async_copy(hp_hbm.at[sidx.at[j]], rows[b], sems[b])

  def pipe(n, prefired):
    if not prefired:
      gather(0, 0)
      gather(1, 1)

    def step(j, b, fire):
      pltpu.make_async_copy(hp_hbm.at[sidx.at[j]], rows[b], sems[b]).wait()
      pltpu.sync_copy(rows[b], acc_sh.at[didx.at[j]], add=True)
      if fire:
        gather(j + 2, b)

    def outer(i, carry):
      step(2 * i, 0, True)
      step(2 * i + 1, 1, True)
      return carry
    lax.fori_loop(0, (n - 2) // 2, outer, 0)
    for b in range(2):
      step(n - 2 + b, b, False)

  pltpu.async_copy(ei3_hbm.at[0, pl.ds(rbase, PH_A)], sidx, sem_i)
  pltpu.async_copy(ei3_hbm.at[1, pl.ds(rbase, PH_A)], didx, sem_i)
  _zero_2d(rows0, CHUNK)
  r0 = s * RPT_ACC
  for k in range(RPT_ACC // CHUNK):
    pltpu.async_copy(rows0, acc_sh.at[pl.ds(r0 + k * CHUNK, CHUNK)], sem_d)
  for k in range(RPT_ACC // CHUNK):
    pltpu.make_async_copy(rows0, acc_sh.at[pl.ds(r0, CHUNK)], sem_d).wait()
  pltpu.make_async_copy(ei3_hbm.at[0, pl.ds(rbase, PH_A)], sidx, sem_i).wait()
  pltpu.make_async_copy(ei3_hbm.at[1, pl.ds(rbase, PH_A)], didx, sem_i).wait()
  gather(0, 0)
  gather(1, 1)
  plsc.subcore_barrier()

  pipe(PH_A, prefired=True)

  rb2 = rbase + PH_A
  pltpu.async_copy(ei3_hbm.at[0, pl.ds(rb2, 16)], sidx.at[pl.ds(0, 16)],
                   sem_i)
  pltpu.async_copy(ei3_hbm.at[1, pl.ds(rb2, 16)], didx.at[pl.ds(0, 16)],
                   sem_i)

  @pl.when(wid < BIG)
  def _more8():
    pltpu.sync_copy(ei3_hbm.at[0, pl.ds(rb2 + 16, 8)], sidx.at[pl.ds(16, 8)])
    pltpu.sync_copy(ei3_hbm.at[1, pl.ds(rb2 + 16, 8)], didx.at[pl.ds(16, 8)])

  @pl.when(wid == NW - 1)
  def _more4():
    pltpu.sync_copy(ei3_hbm.at[0, pl.ds(rb2 + 16, 4)], sidx.at[pl.ds(16, 4)])
    pltpu.sync_copy(ei3_hbm.at[1, pl.ds(rb2 + 16, 4)], didx.at[pl.ds(16, 4)])

  pltpu.make_async_copy(ei3_hbm.at[0, pl.ds(rb2, 16)], sidx.at[pl.ds(0, 16)],
                        sem_i).wait()
  pltpu.make_async_copy(ei3_hbm.at[1, pl.ds(rb2, 16)], didx.at[pl.ds(0, 16)],
                        sem_i).wait()
  pipe(ntot - PH_A, prefired=False)

  plsc.subcore_barrier()

  for k in range(RPT_ACC // CHUNK):
    b = k % 2
    if k >= 2:
      pltpu.make_async_copy(rows[b], out_hbm.at[c, pl.ds(0, CHUNK)],
                            sem_d).wait()
    pltpu.sync_copy(acc_sh.at[pl.ds(r0 + k * CHUNK, CHUNK)], rows[b])
    pltpu.async_copy(rows[b], out_hbm.at[c, pl.ds(r0 + k * CHUNK, CHUNK)],
                     sem_d)
  for b in range(2):
    pltpu.make_async_copy(rows[b], out_hbm.at[c, pl.ds(0, CHUNK)],
                          sem_d).wait()


def _k2_body(x_ref, w_ref, d0_ref, d1_ref, hp_ref, dinv_ref):
  h = jnp.dot(x_ref[...], w_ref[...],
              preferred_element_type=jnp.float32,
              precision=lax.Precision.HIGHEST)
  deg = d0_ref[...] + d1_ref[...] + 1.0
  dinv = lax.rsqrt(deg)
  hp_ref[...] = h * dinv.reshape(N_NODES, 1)
  dinv_ref[...] = dinv


def _k2_call(x, W1, d0, d1):
  return pl.pallas_call(
      _k2_body,
      out_shape=[
          jax.ShapeDtypeStruct((N_NODES, D), jnp.float32),
          jax.ShapeDtypeStruct((N_NODES,), jnp.float32),
      ],
  )(x, W1, d0, d1)


_R4 = 5000
_NB4 = N_NODES // _R4


def _k4_body(acc_ref, hp_ref, dinv_ref, batch_ref, b1_ref, wl_ref, bl_ref,
             out_ref, sums_sc, cnt_sc):
  i = pl.program_id(0)

  @pl.when(i == 0)
  def _init():
    sums_sc[...] = jnp.zeros_like(sums_sc)
    cnt_sc[...] = jnp.zeros_like(cnt_sc)

  acc = acc_ref[0] + acc_ref[1]
  dinv = dinv_ref[pl.ds(i, 1), :]
  node = dinv.reshape(_R4, 1) * (acc + hp_ref[...])
  node = jnp.maximum(node + b1_ref[...], 0.0)
  onehot_t = (batch_ref[pl.ds(i, 1), :] ==
              lax.broadcasted_iota(jnp.int32, (G, 1), 0)).astype(jnp.float32)
  sums_sc[...] += lax.dot_general(
      onehot_t, node, (((1,), (0,)), ((), ())),
      preferred_element_type=jnp.float32, precision=lax.Precision.HIGHEST)
  cnt_sc[...] += lax.dot_general(
      onehot_t, jnp.ones((_R4, 1), jnp.float32), (((1,), (0,)), ((), ())),
      preferred_element_type=jnp.float32, precision=lax.Precision.HIGHEST)

  @pl.when(i == _NB4 - 1)
  def _fin():
    pooled = sums_sc[...] / jnp.maximum(cnt_sc[...], 1.0)
    out_ref[...] = jnp.dot(pooled, wl_ref[...],
                           preferred_element_type=jnp.float32,
                           precision=lax.Precision.HIGHEST) + bl_ref[...]


def _k4_call(acc, hp, dinv, batch1d, b1, Wl, bl):
  return pl.pallas_call(
      _k4_body,
      grid=(_NB4,),
      in_specs=[
          pl.BlockSpec((NC, _R4, D), lambda i: (0, i, 0)),
          pl.BlockSpec((_R4, D), lambda i: (i, 0)),
          pl.BlockSpec((_NB4, _R4), lambda i: (0, 0)),
          pl.BlockSpec((_NB4, _R4), lambda i: (0, 0)),
          pl.BlockSpec((1, D), lambda i: (0, 0)),
          pl.BlockSpec((D, D), lambda i: (0, 0)),
          pl.BlockSpec((1, D), lambda i: (0, 0)),
      ],
      out_specs=pl.BlockSpec((G, D), lambda i: (0, 0)),
      out_shape=jax.ShapeDtypeStruct((G, D), jnp.float32),
      scratch_shapes=[
          pltpu.VMEM((G, D), jnp.float32),
          pltpu.VMEM((G, 1), jnp.float32),
      ],
      compiler_params=pltpu.CompilerParams(
          dimension_semantics=("arbitrary",)),
  )(acc, hp, dinv.reshape(_NB4, _R4), batch1d.reshape(_NB4, _R4),
    b1, Wl, bl)


@jax.jit
def kernel(x, edge_index, batch, W1, b1, Wl, bl):
  ei3 = edge_index.reshape(2, EROWS, CHUNK)
  deg_flat = _deg_kernel(ei3)
  d0 = deg_flat[:N_NODES]
  d1 = deg_flat[N_PAD:N_PAD + N_NODES]
  hp, dinv = _k2_call(x, W1, d0, d1)
  acc = _msg_kernel(ei3, hp)
  out = _k4_call(acc, hp, dinv, batch,
                 b1.reshape(1, D), Wl, bl.reshape(1, D))
  return out
